# baseline (speedup 1.0000x reference)
import functools

import jax
import jax.numpy as jnp
from jax import lax
from jax.experimental import pallas as pl
from jax.experimental.pallas import tpu as pltpu
from jax.experimental.pallas import tpu_sc as plsc

N = 50000
E = 800000
D_IN = 300
H = 64

NC = 2
NS = 16
LANES = 16

ROUNDS = 4
NPART = NC * ROUNDS
PSIZE = 6256
ACC_ROWS = PSIZE + LANES
OUT_ROWS = NPART * PSIZE
ETILE = E // NS
CHUNK = 2000
NCHUNK = ETILE // CHUNK
NGRP = CHUNK // LANES
SUB = 64
NZ = ACC_ROWS // NS
OW = 400
OW_LAST = PSIZE - (NS - 1) * OW


def _sc_mesh():
  return plsc.VectorSubcoreMesh(core_axis_name="c", subcore_axis_name="s")


@functools.partial(
    pl.kernel,
    out_type=jax.ShapeDtypeStruct((OUT_ROWS, 2 * H), jnp.float32),
    mesh=_sc_mesh(),
    scratch_types=dict(
        sbuf0=pltpu.VMEM((CHUNK,), jnp.int32),
        dbuf0=pltpu.VMEM((CHUNK,), jnp.int32),
        sbuf1=pltpu.VMEM((CHUNK,), jnp.int32),
        dbuf1=pltpu.VMEM((CHUNK,), jnp.int32),
        csrc=pltpu.VMEM((CHUNK + 2 * SUB,), jnp.int32),
        cdst=pltpu.VMEM((CHUNK + 2 * SUB,), jnp.int32),
        idxsA=pltpu.VMEM((SUB,), jnp.int32),
        idxdA=pltpu.VMEM((SUB,), jnp.int32),
        idxcA=pltpu.VMEM((2 * SUB,), jnp.int32),
        g1A=pltpu.VMEM((SUB, 2 * H), jnp.float32),
        g2A=pltpu.VMEM((2 * SUB, 2 * H), jnp.float32),
        idxsB=pltpu.VMEM((SUB,), jnp.int32),
        idxdB=pltpu.VMEM((SUB,), jnp.int32),
        idxcB=pltpu.VMEM((2 * SUB,), jnp.int32),
        g1B=pltpu.VMEM((SUB, 2 * H), jnp.float32),
        g2B=pltpu.VMEM((2 * SUB, 2 * H), jnp.float32),
        obufA=pltpu.VMEM((SUB, 2 * H), jnp.float32),
        obufB=pltpu.VMEM((SUB, 2 * H), jnp.float32),
        sdixA=pltpu.VMEM((SUB,), jnp.int32),
        sdixB=pltpu.VMEM((SUB,), jnp.int32),
        acc=pltpu.VMEM_SHARED((ACC_ROWS, 2 * H), jnp.float32),
        semA=pltpu.SemaphoreType.DMA,
        semB=pltpu.SemaphoreType.DMA,
        ssemA=pltpu.SemaphoreType.DMA,
        ssemB=pltpu.SemaphoreType.DMA,
        csem0=pltpu.SemaphoreType.DMA,
        csem1=pltpu.SemaphoreType.DMA,
    ),
    compiler_params=pltpu.CompilerParams(needs_layout_passes=False),
)
def _sc_edges(src_hbm, dst_hbm, thc_hbm, tf_hbm, zeros_hbm, out_hbm,
              sbuf0, dbuf0, sbuf1, dbuf1, csrc, cdst,
              idxsA, idxdA, idxcA, g1A, g2A,
              idxsB, idxdB, idxcB, g1B, g2B,
              obufA, obufB, sdixA, sdixB, acc, semA, semB, ssemA, ssemB,
              csem0, csem1):
  cid = lax.axis_index("c")
  sid = lax.axis_index("s")
  base_e = sid * ETILE
  iota = lax.iota(jnp.int32, LANES)
  bufA = (idxsA, idxdA, idxcA, g1A, g2A, semA, obufA, sdixA, ssemA)
  bufB = (idxsB, idxdB, idxcB, g1B, g2B, semB, obufB, sdixB, ssemB)
  ebufs = ((sbuf0, dbuf0, csem0), (sbuf1, dbuf1, csem1))

  def fill_and_start(buf, jb, lo):
    idxs, idxd, idxc, g1, g2, sem, obuf, sdix, ssem = buf
    for i in range(SUB // LANES):
      sl = pl.ds(i * LANES, LANES)
      vs = csrc[pl.ds(jb + i * LANES, LANES)]
      vd = cdst[pl.ds(jb + i * LANES, LANES)]
      idxs[sl] = vs
      idxd[sl] = vd
      idxc[sl] = vs
      idxc[pl.ds(SUB + i * LANES, LANES)] = jnp.minimum(vd + lo, N - 1)
    pltpu.make_async_copy(thc_hbm.at[idxs], g1, sem).start()
    pltpu.make_async_copy(tf_hbm.at[idxc], g2, sem).start()

  def wait_scatter(buf):
    _, _, _, _, _, _, obuf, sdix, ssem = buf
    pltpu.make_async_copy(obuf, acc.at[sdix], ssem).wait()

  def finish_sub(buf, j):
    idxs, idxd, idxc, g1, g2, sem, obuf, sdix, ssem = buf
    pltpu.make_async_copy(thc_hbm.at[idxs], g1, sem).wait()
    pltpu.make_async_copy(tf_hbm.at[idxc], g2, sem).wait()

    @pl.when(j >= 2)
    def _():
      wait_scatter(buf)

    def row(r0, _):
      for q in range(H // LANES):
        slh = pl.ds(q * LANES, LANES)
        slc = pl.ds(H + q * LANES, LANES)
        obuf[r0, slh] = g1[r0, slh]
        z = g2[r0, slh] + g2[SUB + r0, slc]
        f = 1.0 / (1.0 + jnp.exp(-z))
        obuf[r0, slc] = f * g1[r0, slc]
      return 0

    lax.fori_loop(0, SUB, row, 0)
    for i in range(SUB // LANES):
      sl = pl.ds(i * LANES, LANES)
      sdix[sl] = idxd[sl]
    pltpu.make_async_copy(obuf, acc.at[sdix], ssem).start(add=True)

  def flush(nfull, off, lo):
    @pl.when(nfull > 0)
    def _():
      fill_and_start(bufA, 0, lo)

    def do_sub(j, cur, nxt):
      @pl.when(j + 1 < nfull)
      def _():
        fill_and_start(nxt, (j + 1) * SUB, lo)
      finish_sub(cur, j)

    def pair(i, _):
      do_sub(2 * i, bufA, bufB)
      do_sub(2 * i + 1, bufB, bufA)
      return 0

    lax.fori_loop(0, nfull // 2, pair, 0)

    @pl.when(nfull % 2 == 1)
    def _():
      do_sub(nfull - 1, bufA, bufB)

    @pl.when((nfull >= 1) & ((nfull - 1) % 2 == 0))
    def _():
      wait_scatter(bufA)

    @pl.when((nfull >= 1) & ((nfull - 1) % 2 == 1))
    def _():
      wait_scatter(bufB)

    @pl.when((nfull >= 2) & (nfull % 2 == 0))
    def _():
      wait_scatter(bufA)

    @pl.when((nfull >= 2) & (nfull % 2 == 1))
    def _():
      wait_scatter(bufB)

    rem_base = nfull * SUB
    for i in range(SUB // LANES):
      v1 = csrc[pl.ds(rem_base + i * LANES, LANES)]
      v2 = cdst[pl.ds(rem_base + i * LANES, LANES)]
      csrc[pl.ds(i * LANES, LANES)] = v1
      cdst[pl.ds(i * LANES, LANES)] = v2
    return off - rem_base

  def start_echunk(ch, eb):
    sbuf, dbuf, csem = eb
    off0 = pl.multiple_of(base_e + ch * CHUNK, 8)
    pltpu.make_async_copy(src_hbm.at[pl.ds(off0, CHUNK)], sbuf, csem).start()
    pltpu.make_async_copy(dst_hbm.at[pl.ds(off0, CHUNK)], dbuf, csem).start()

  def scan_chunk(ch, off, eb, lo):
    sbuf, dbuf, csem = eb
    off0 = pl.multiple_of(base_e + ch * CHUNK, 8)
    pltpu.make_async_copy(src_hbm.at[pl.ds(off0, CHUNK)], sbuf, csem).wait()
    pltpu.make_async_copy(dst_hbm.at[pl.ds(off0, CHUNK)], dbuf, csem).wait()

    def grp(g, off):
      gb = g * LANES
      s16 = sbuf[pl.ds(gb, LANES)]
      dl = dbuf[pl.ds(gb, LANES)] - lo
      m = (dl >= 0) & (dl < PSIZE)
      cnt = jnp.sum(m.astype(jnp.int32))
      plsc.store_compressed(csrc.at[pl.ds(off, LANES)], s16, mask=m)
      plsc.store_compressed(cdst.at[pl.ds(off, LANES)], dl, mask=m)
      return off + cnt

    return lax.fori_loop(0, NGRP, grp, off)

  def round_body(rr, _):
    p = cid * ROUNDS + rr
    lo = p * PSIZE
    pltpu.sync_copy(zeros_hbm.at[pl.ds(sid * NZ, NZ)],
                    acc.at[pl.ds(sid * NZ, NZ)])
    plsc.subcore_barrier()

    start_echunk(0, ebufs[0])

    def chunk_pair(i, off):
      ch = 2 * i
      start_echunk(ch + 1, ebufs[1])
      off = scan_chunk(ch, off, ebufs[0], lo)
      off = flush(off // SUB, off, lo)

      @pl.when(ch + 2 < NCHUNK)
      def _():
        start_echunk(ch + 2, ebufs[0])
      off = scan_chunk(ch + 1, off, ebufs[1], lo)
      return flush(off // SUB, off, lo)

    off = lax.fori_loop(0, NCHUNK // 2, chunk_pair, 0)
    off = scan_chunk(NCHUNK - 1, off, ebufs[0], lo)
    off = flush(off // SUB, off, lo)

    for i in range(SUB // LANES):
      csrc[pl.ds(off + i * LANES, LANES)] = iota + i * LANES
      cdst[pl.ds(off + i * LANES, LANES)] = iota + PSIZE
    flush((off + SUB - 1) // SUB, 0, lo)

    plsc.subcore_barrier()
    base_o = pl.multiple_of(p * PSIZE, 8)

    @pl.when(sid < NS - 1)
    def _():
      o = pl.multiple_of(sid * OW, 8)
      pltpu.sync_copy(acc.at[pl.ds(o, OW)], out_hbm.at[pl.ds(base_o + o, OW)])

    @pl.when(sid == NS - 1)
    def _():
      o = (NS - 1) * OW
      pltpu.sync_copy(acc.at[pl.ds(o, OW_LAST)],
                      out_hbm.at[pl.ds(base_o + o, OW_LAST)])

    plsc.subcore_barrier()
    return 0

  lax.fori_loop(0, ROUNDS, round_body, 0)


_ROWB = 5000
_GRID = N // _ROWB


def _pre_body(x_ref, cc_ref, ch_ref, wioux_ref, bsum_ref, wfx_ref, bfx_ref,
              wfh_ref, bfh_ref, xw_ref, thc_ref, tf_ref):
  x = x_ref[...]
  ch = ch_ref[...]
  xw_ref[...] = (
      jnp.dot(x, wioux_ref[...], preferred_element_type=jnp.float32)
      + bsum_ref[...])
  thc_ref[...] = jnp.concatenate([ch, cc_ref[...]], axis=1)
  fh = jnp.dot(ch, wfh_ref[...], preferred_element_type=jnp.float32) + bfh_ref[...]
  fx = jnp.dot(x, wfx_ref[...], preferred_element_type=jnp.float32) + bfx_ref[...]
  tf_ref[...] = jnp.concatenate([fh, fx], axis=1)


def _pre(x, child_c, child_h, W_ioux, bsum, W_fx, bfx, W_fh, bfh):
  return pl.pallas_call(
      _pre_body,
      grid=(_GRID,),
      in_specs=[
          pl.BlockSpec((_ROWB, D_IN), lambda i: (i, 0)),
          pl.BlockSpec((_ROWB, H), lambda i: (i, 0)),
          pl.BlockSpec((_ROWB, H), lambda i: (i, 0)),
          pl.BlockSpec((D_IN, 3 * H), lambda i: (0, 0)),
          pl.BlockSpec((1, 3 * H), lambda i: (0, 0)),
          pl.BlockSpec((D_IN, H), lambda i: (0, 0)),
          pl.BlockSpec((1, H), lambda i: (0, 0)),
          pl.BlockSpec((H, H), lambda i: (0, 0)),
          pl.BlockSpec((1, H), lambda i: (0, 0)),
      ],
      out_specs=[
          pl.BlockSpec((_ROWB, 3 * H), lambda i: (i, 0)),
          pl.BlockSpec((_ROWB, 2 * H), lambda i: (i, 0)),
          pl.BlockSpec((_ROWB, 2 * H), lambda i: (i, 0)),
      ],
      out_shape=[
          jax.ShapeDtypeStruct((N, 3 * H), jnp.float32),
          jax.ShapeDtypeStruct((N, 2 * H), jnp.float32),
          jax.ShapeDtypeStruct((N, 2 * H), jnp.float32),
      ],
  )(x, child_c, child_h, W_ioux, bsum, W_fx, bfx, W_fh, bfh)


def _post_body(hf_ref, xw_ref, wiouh_ref, c_ref, h_ref):
  hf = hf_ref[...]
  iou = xw_ref[...] + jnp.dot(
      hf[:, 0:H], wiouh_ref[...], preferred_element_type=jnp.float32)
  i = jax.nn.sigmoid(iou[:, 0:H])
  o = jax.nn.sigmoid(iou[:, H:2 * H])
  u = jnp.tanh(iou[:, 2 * H:3 * H])
  c = i * u + hf[:, H:2 * H]
  c_ref[...] = c
  h_ref[...] = o * jnp.tanh(c)


def _post(hf, xw, W_iouh):
  return pl.pallas_call(
      _post_body,
      grid=(_GRID,),
      in_specs=[
          pl.BlockSpec((_ROWB, 2 * H), lambda i: (i, 0)),
          pl.BlockSpec((_ROWB, 3 * H), lambda i: (i, 0)),
          pl.BlockSpec((H, 3 * H), lambda i: (0, 0)),
      ],
      out_specs=[
          pl.BlockSpec((_ROWB, H), lambda i: (i, 0)),
          pl.BlockSpec((_ROWB, H), lambda i: (i, 0)),
      ],
      out_shape=[
          jax.ShapeDtypeStruct((N, H), jnp.float32),
          jax.ShapeDtypeStruct((N, H), jnp.float32),
      ],
  )(hf, xw, W_iouh)


def kernel(x, edge_index, child_c, child_h,
           W_ioux, b_ioux, W_iouh, b_iouh, W_fx, b_fx, W_fh, b_fh):
  src = edge_index[0]
  dst = edge_index[1]
  bsum = (b_ioux + b_iouh).reshape(1, 3 * H)
  xw, thc, tf = _pre(x, child_c, child_h, W_ioux, bsum,
                     W_fx, b_fx.reshape(1, H), W_fh, b_fh.reshape(1, H))
  zeros_acc = jnp.zeros((ACC_ROWS, 2 * H), jnp.float32)
  hf = _sc_edges(src, dst, thc, tf, zeros_acc)
  c, h = _post(hf, xw, W_iouh)
  return (c, h)


# speedup vs baseline: 1.1672x; 1.1672x over previous
import functools

import jax
import jax.numpy as jnp
from jax import lax
from jax.experimental import pallas as pl
from jax.experimental.pallas import tpu as pltpu
from jax.experimental.pallas import tpu_sc as plsc

N = 50000
E = 800000
D_IN = 300
H = 64

NC = 2
NS = 16
LANES = 16

ROUNDS = 4
NPART = NC * ROUNDS
PSIZE = 6256
ACC_ROWS = PSIZE + LANES
OUT_ROWS = NPART * PSIZE
ETILE = E // NS
CHUNK = 2000
NCHUNK = ETILE // CHUNK
NGRP = CHUNK // LANES
SUB = 64
NZ = ACC_ROWS // NS
OW = 400
OW_LAST = PSIZE - (NS - 1) * OW


def _sc_mesh():
  return plsc.VectorSubcoreMesh(core_axis_name="c", subcore_axis_name="s")


@functools.partial(
    pl.kernel,
    out_type=jax.ShapeDtypeStruct((OUT_ROWS, 2 * H), jnp.float32),
    mesh=_sc_mesh(),
    scratch_types=dict(
        sbuf0=pltpu.VMEM((CHUNK,), jnp.int32),
        dbuf0=pltpu.VMEM((CHUNK,), jnp.int32),
        sbuf1=pltpu.VMEM((CHUNK,), jnp.int32),
        dbuf1=pltpu.VMEM((CHUNK,), jnp.int32),
        csrc=pltpu.VMEM((CHUNK + 2 * SUB,), jnp.int32),
        cdst=pltpu.VMEM((CHUNK + 2 * SUB,), jnp.int32),
        idxdA=pltpu.VMEM((SUB,), jnp.int32),
        idxcA=pltpu.VMEM((2 * SUB,), jnp.int32),
        gA=pltpu.VMEM((2 * SUB, 2 * H), jnp.int32),
        idxdB=pltpu.VMEM((SUB,), jnp.int32),
        idxcB=pltpu.VMEM((2 * SUB,), jnp.int32),
        gB=pltpu.VMEM((2 * SUB, 2 * H), jnp.int32),
        obufA=pltpu.VMEM((SUB, 2 * H), jnp.float32),
        obufB=pltpu.VMEM((SUB, 2 * H), jnp.float32),
        sdixA=pltpu.VMEM((SUB,), jnp.int32),
        sdixB=pltpu.VMEM((SUB,), jnp.int32),
        acc=pltpu.VMEM_SHARED((ACC_ROWS, 2 * H), jnp.float32),
        semA=pltpu.SemaphoreType.DMA,
        semB=pltpu.SemaphoreType.DMA,
        ssemA=pltpu.SemaphoreType.DMA,
        ssemB=pltpu.SemaphoreType.DMA,
        csem0=pltpu.SemaphoreType.DMA,
        csem1=pltpu.SemaphoreType.DMA,
    ),
    compiler_params=pltpu.CompilerParams(needs_layout_passes=False),
)
def _sc_edges(src_hbm, dst_hbm, tab_hbm, zeros_hbm, out_hbm,
              sbuf0, dbuf0, sbuf1, dbuf1, csrc, cdst,
              idxdA, idxcA, gA, idxdB, idxcB, gB,
              obufA, obufB, sdixA, sdixB, acc, semA, semB, ssemA, ssemB,
              csem0, csem1):
  cid = lax.axis_index("c")
  sid = lax.axis_index("s")
  base_e = sid * ETILE
  iota = lax.iota(jnp.int32, LANES)
  bufA = (idxdA, idxcA, gA, semA, obufA, sdixA, ssemA)
  bufB = (idxdB, idxcB, gB, semB, obufB, sdixB, ssemB)
  ebufs = ((sbuf0, dbuf0, csem0), (sbuf1, dbuf1, csem1))

  def fill_and_start(buf, jb, lo):
    idxd, idxc, g, sem, obuf, sdix, ssem = buf
    for i in range(SUB // LANES):
      sl = pl.ds(i * LANES, LANES)
      vs = csrc[pl.ds(jb + i * LANES, LANES)]
      vd = cdst[pl.ds(jb + i * LANES, LANES)]
      idxd[sl] = vd
      idxc[sl] = vs
      idxc[pl.ds(SUB + i * LANES, LANES)] = jnp.minimum(vd + lo, N - 1)
    pltpu.make_async_copy(tab_hbm.at[idxc], g, sem).start()

  def wait_scatter(buf):
    _, _, _, _, obuf, sdix, ssem = buf
    pltpu.make_async_copy(obuf, acc.at[sdix], ssem).wait()

  def finish_sub(buf, j):
    idxd, idxc, g, sem, obuf, sdix, ssem = buf
    pltpu.make_async_copy(tab_hbm.at[idxc], g, sem).wait()

    @pl.when(j >= 2)
    def _():
      wait_scatter(buf)

    maskhi = jnp.full((LANES,), -65536, jnp.int32)

    def row(r0, _):
      for q in range(H // 32):
        wh = g[r0, pl.ds(q * LANES, LANES)]
        wc = g[r0, pl.ds(32 + q * LANES, LANES)]
        wfh = g[r0, pl.ds(64 + q * LANES, LANES)]
        wfx = g[SUB + r0, pl.ds(96 + q * LANES, LANES)]
        z_lo = (plsc.bitcast(wfh << 16, jnp.float32)
                + plsc.bitcast(wfx << 16, jnp.float32))
        z_hi = (plsc.bitcast(wfh & maskhi, jnp.float32)
                + plsc.bitcast(wfx & maskhi, jnp.float32))
        f_lo = 1.0 / (1.0 + jnp.exp(-z_lo))
        f_hi = 1.0 / (1.0 + jnp.exp(-z_hi))
        b = q * 32
        obuf[r0, pl.ds(b, LANES)] = plsc.bitcast(wh << 16, jnp.float32)
        obuf[r0, pl.ds(b + 16, LANES)] = plsc.bitcast(wh & maskhi,
                                                      jnp.float32)
        obuf[r0, pl.ds(H + b, LANES)] = f_lo * plsc.bitcast(
            wc << 16, jnp.float32)
        obuf[r0, pl.ds(H + b + 16, LANES)] = f_hi * plsc.bitcast(
            wc & maskhi, jnp.float32)
      return 0

    lax.fori_loop(0, SUB, row, 0)
    for i in range(SUB // LANES):
      sl = pl.ds(i * LANES, LANES)
      sdix[sl] = idxd[sl]
    pltpu.make_async_copy(obuf, acc.at[sdix], ssem).start(add=True)

  def flush(nfull, off, lo):
    @pl.when(nfull > 0)
    def _():
      fill_and_start(bufA, 0, lo)

    def do_sub(j, cur, nxt):
      @pl.when(j + 1 < nfull)
      def _():
        fill_and_start(nxt, (j + 1) * SUB, lo)
      finish_sub(cur, j)

    def pair(i, _):
      do_sub(2 * i, bufA, bufB)
      do_sub(2 * i + 1, bufB, bufA)
      return 0

    lax.fori_loop(0, nfull // 2, pair, 0)

    @pl.when(nfull % 2 == 1)
    def _():
      do_sub(nfull - 1, bufA, bufB)

    @pl.when((nfull >= 1) & ((nfull - 1) % 2 == 0))
    def _():
      wait_scatter(bufA)

    @pl.when((nfull >= 1) & ((nfull - 1) % 2 == 1))
    def _():
      wait_scatter(bufB)

    @pl.when((nfull >= 2) & (nfull % 2 == 0))
    def _():
      wait_scatter(bufA)

    @pl.when((nfull >= 2) & (nfull % 2 == 1))
    def _():
      wait_scatter(bufB)

    rem_base = nfull * SUB
    for i in range(SUB // LANES):
      v1 = csrc[pl.ds(rem_base + i * LANES, LANES)]
      v2 = cdst[pl.ds(rem_base + i * LANES, LANES)]
      csrc[pl.ds(i * LANES, LANES)] = v1
      cdst[pl.ds(i * LANES, LANES)] = v2
    return off - rem_base

  def start_echunk(ch, eb):
    sbuf, dbuf, csem = eb
    off0 = pl.multiple_of(base_e + ch * CHUNK, 8)
    pltpu.make_async_copy(src_hbm.at[pl.ds(off0, CHUNK)], sbuf, csem).start()
    pltpu.make_async_copy(dst_hbm.at[pl.ds(off0, CHUNK)], dbuf, csem).start()

  def scan_chunk(ch, off, eb, lo):
    sbuf, dbuf, csem = eb
    off0 = pl.multiple_of(base_e + ch * CHUNK, 8)
    pltpu.make_async_copy(src_hbm.at[pl.ds(off0, CHUNK)], sbuf, csem).wait()
    pltpu.make_async_copy(dst_hbm.at[pl.ds(off0, CHUNK)], dbuf, csem).wait()

    def grp(g, off):
      gb = g * LANES
      s16 = sbuf[pl.ds(gb, LANES)]
      dl = dbuf[pl.ds(gb, LANES)] - lo
      m = (dl >= 0) & (dl < PSIZE)
      cnt = jnp.sum(m.astype(jnp.int32))
      plsc.store_compressed(csrc.at[pl.ds(off, LANES)], s16, mask=m)
      plsc.store_compressed(cdst.at[pl.ds(off, LANES)], dl, mask=m)
      return off + cnt

    return lax.fori_loop(0, NGRP, grp, off)

  def round_body(rr, _):
    p = cid * ROUNDS + rr
    lo = p * PSIZE
    pltpu.sync_copy(zeros_hbm.at[pl.ds(sid * NZ, NZ)],
                    acc.at[pl.ds(sid * NZ, NZ)])
    plsc.subcore_barrier()

    start_echunk(0, ebufs[0])

    def chunk_pair(i, off):
      ch = 2 * i
      start_echunk(ch + 1, ebufs[1])
      off = scan_chunk(ch, off, ebufs[0], lo)
      off = flush(off // SUB, off, lo)

      @pl.when(ch + 2 < NCHUNK)
      def _():
        start_echunk(ch + 2, ebufs[0])
      off = scan_chunk(ch + 1, off, ebufs[1], lo)
      return flush(off // SUB, off, lo)

    off = lax.fori_loop(0, NCHUNK // 2, chunk_pair, 0)
    off = scan_chunk(NCHUNK - 1, off, ebufs[0], lo)
    off = flush(off // SUB, off, lo)

    for i in range(SUB // LANES):
      csrc[pl.ds(off + i * LANES, LANES)] = iota + i * LANES
      cdst[pl.ds(off + i * LANES, LANES)] = iota + PSIZE
    flush((off + SUB - 1) // SUB, 0, lo)

    plsc.subcore_barrier()
    base_o = pl.multiple_of(p * PSIZE, 8)

    @pl.when(sid < NS - 1)
    def _():
      o = pl.multiple_of(sid * OW, 8)
      pltpu.sync_copy(acc.at[pl.ds(o, OW)], out_hbm.at[pl.ds(base_o + o, OW)])

    @pl.when(sid == NS - 1)
    def _():
      o = (NS - 1) * OW
      pltpu.sync_copy(acc.at[pl.ds(o, OW_LAST)],
                      out_hbm.at[pl.ds(base_o + o, OW_LAST)])

    plsc.subcore_barrier()
    return 0

  lax.fori_loop(0, ROUNDS, round_body, 0)


_ROWB = 5000
_GRID = N // _ROWB


def _pack64(x):
  lo = jnp.concatenate([x[:, 0:16], x[:, 32:48]], axis=1)
  hi = jnp.concatenate([x[:, 16:32], x[:, 48:64]], axis=1)
  ulo = lax.bitcast_convert_type(lo.astype(jnp.bfloat16), jnp.uint16)
  uhi = lax.bitcast_convert_type(hi.astype(jnp.bfloat16), jnp.uint16)
  return ulo.astype(jnp.int32) | (uhi.astype(jnp.int32) << 16)


def _pre_body(x_ref, cc_ref, ch_ref, wioux_ref, bsum_ref, wfx_ref, bfx_ref,
              wfh_ref, bfh_ref, xw_ref, tab_ref):
  x = x_ref[...]
  ch = ch_ref[...]
  xw_ref[...] = (
      jnp.dot(x, wioux_ref[...], preferred_element_type=jnp.float32)
      + bsum_ref[...])
  fh = jnp.dot(ch, wfh_ref[...], preferred_element_type=jnp.float32) + bfh_ref[...]
  fx = jnp.dot(x, wfx_ref[...], preferred_element_type=jnp.float32) + bfx_ref[...]
  tab_ref[...] = jnp.concatenate(
      [_pack64(ch), _pack64(cc_ref[...]), _pack64(fh), _pack64(fx)], axis=1)


def _pre(x, child_c, child_h, W_ioux, bsum, W_fx, bfx, W_fh, bfh):
  return pl.pallas_call(
      _pre_body,
      grid=(_GRID,),
      in_specs=[
          pl.BlockSpec((_ROWB, D_IN), lambda i: (i, 0)),
          pl.BlockSpec((_ROWB, H), lambda i: (i, 0)),
          pl.BlockSpec((_ROWB, H), lambda i: (i, 0)),
          pl.BlockSpec((D_IN, 3 * H), lambda i: (0, 0)),
          pl.BlockSpec((1, 3 * H), lambda i: (0, 0)),
          pl.BlockSpec((D_IN, H), lambda i: (0, 0)),
          pl.BlockSpec((1, H), lambda i: (0, 0)),
          pl.BlockSpec((H, H), lambda i: (0, 0)),
          pl.BlockSpec((1, H), lambda i: (0, 0)),
      ],
      out_specs=[
          pl.BlockSpec((_ROWB, 3 * H), lambda i: (i, 0)),
          pl.BlockSpec((_ROWB, 2 * H), lambda i: (i, 0)),
      ],
      out_shape=[
          jax.ShapeDtypeStruct((N, 3 * H), jnp.float32),
          jax.ShapeDtypeStruct((N, 2 * H), jnp.int32),
      ],
  )(x, child_c, child_h, W_ioux, bsum, W_fx, bfx, W_fh, bfh)


def _post_body(hf_ref, xw_ref, wiouh_ref, c_ref, h_ref):
  hf = hf_ref[...]
  iou = xw_ref[...] + jnp.dot(
      hf[:, 0:H], wiouh_ref[...], preferred_element_type=jnp.float32)
  i = jax.nn.sigmoid(iou[:, 0:H])
  o = jax.nn.sigmoid(iou[:, H:2 * H])
  u = jnp.tanh(iou[:, 2 * H:3 * H])
  c = i * u + hf[:, H:2 * H]
  c_ref[...] = c
  h_ref[...] = o * jnp.tanh(c)


def _post(hf, xw, W_iouh):
  return pl.pallas_call(
      _post_body,
      grid=(_GRID,),
      in_specs=[
          pl.BlockSpec((_ROWB, 2 * H), lambda i: (i, 0)),
          pl.BlockSpec((_ROWB, 3 * H), lambda i: (i, 0)),
          pl.BlockSpec((H, 3 * H), lambda i: (0, 0)),
      ],
      out_specs=[
          pl.BlockSpec((_ROWB, H), lambda i: (i, 0)),
          pl.BlockSpec((_ROWB, H), lambda i: (i, 0)),
      ],
      out_shape=[
          jax.ShapeDtypeStruct((N, H), jnp.float32),
          jax.ShapeDtypeStruct((N, H), jnp.float32),
      ],
  )(hf, xw, W_iouh)


def kernel(x, edge_index, child_c, child_h,
           W_ioux, b_ioux, W_iouh, b_iouh, W_fx, b_fx, W_fh, b_fh):
  src = edge_index[0]
  dst = edge_index[1]
  bsum = (b_ioux + b_iouh).reshape(1, 3 * H)
  xw, tab = _pre(x, child_c, child_h, W_ioux, bsum,
                 W_fx, b_fx.reshape(1, H), W_fh, b_fh.reshape(1, H))
  zeros_acc = jnp.zeros((ACC_ROWS, 2 * H), jnp.float32)
  hf = _sc_edges(src, dst, tab, zeros_acc)
  c, h = _post(hf, xw, W_iouh)
  return (c, h)
